# baseline (speedup 1.0000x reference)
import functools
import math

import jax
import jax.numpy as jnp
from jax import lax
from jax.experimental import pallas as pl
from jax.experimental.pallas import tpu as pltpu
from jax.experimental.pallas import tpu_sc as plsc

ALPHA = 0.5
NC = 2
NS = 16
C = 80


def _pad_rows(n):
    per = -(-n // NS)
    per = -(-per // 8) * 8
    return per * NS, per


_NB = 4
_DW = 128


@functools.lru_cache(maxsize=None)
def _make_deg_kernel(n, e):
    ep = e // (NC * NS)
    nchunks = ep // C
    np_, rows = _pad_rows(n)
    mesh = plsc.VectorSubcoreMesh(core_axis_name="c", subcore_axis_name="s")

    @functools.partial(
        pl.kernel,
        out_type=jax.ShapeDtypeStruct((NC, np_, _DW), jnp.float32),
        mesh=mesh,
        scratch_types=[
            pltpu.VMEM_SHARED((np_, _DW), jnp.float32),
            pltpu.VMEM((C, _DW), jnp.float32),
        ] + [pltpu.VMEM((C,), jnp.int32) for _ in range(_NB)]
          + [pltpu.SemaphoreType.DMA for _ in range(_NB)],
    )
    def deg_kernel(dst_ix, zeros1, ones, out, deg_sh, onesbuf, *dbsx):
        dbs, sixs = dbsx[:_NB], dbsx[_NB:]
        c = lax.axis_index("c")
        s = lax.axis_index("s")
        r0 = pl.multiple_of(s * rows, 8)
        pltpu.sync_copy(zeros1.at[pl.ds(r0, rows), :], deg_sh.at[pl.ds(r0, rows), :])
        pltpu.sync_copy(ones, onesbuf)
        plsc.subcore_barrier()
        base = c * (e // NC) + s * ep

        six, ssc = sixs[0], sixs[1]

        def group(g, carry):
            j0 = g * _NB
            for b in range(_NB):
                off = pl.multiple_of(base + (j0 + b) * C, 8)
                pltpu.async_copy(dst_ix.at[pl.ds(off, C)], dbs[b], six)
            for b in range(_NB):
                off = pl.multiple_of(base + (j0 + b) * C, 8)
                pltpu.make_async_copy(dst_ix.at[pl.ds(off, C)], dbs[b], six).wait()
            for b in range(_NB):
                pltpu.async_copy(onesbuf, deg_sh.at[dbs[b]], ssc, add=True)
            for b in range(_NB):
                pltpu.make_async_copy(onesbuf, deg_sh.at[dbs[b]], ssc).wait()
            return carry

        lax.fori_loop(0, nchunks // _NB, group, 0)
        for j in range(nchunks - nchunks % _NB, nchunks):
            off = pl.multiple_of(base + j * C, 8)
            pltpu.sync_copy(dst_ix.at[pl.ds(off, C)], dbs[0])
            pltpu.sync_copy(onesbuf, deg_sh.at[dbs[0]], add=True)
        plsc.subcore_barrier()
        pltpu.sync_copy(deg_sh.at[pl.ds(r0, rows), :], out.at[c, pl.ds(r0, rows), :])

    return deg_kernel


_CA = 80
_NG = 1
_NA = 4


@functools.lru_cache(maxsize=None)
def _make_agg_kernel(n, d, e):
    ep = e // (NC * NS)
    nchunks = ep // _CA
    assert ep % _CA == 0
    np_, rows = _pad_rows(n)
    nbuf = _NG * _NA
    mesh = plsc.VectorSubcoreMesh(core_axis_name="c", subcore_axis_name="s")

    @functools.partial(
        pl.kernel,
        out_type=jax.ShapeDtypeStruct((NC, np_, d), jnp.float32),
        mesh=mesh,
        scratch_types=[
            pltpu.VMEM_SHARED((np_, d), jnp.float32),
        ] + [pltpu.VMEM((_CA,), jnp.int32) for _ in range(2 * nbuf)]
          + [pltpu.VMEM((_CA, d), jnp.float32) for _ in range(nbuf)]
          + [pltpu.SemaphoreType.DMA for _ in range(3 * _NG)],
    )
    def agg_kernel(h, src_ix, dst_ix, zeros, out, agg_sh, *rest):
        sbs = rest[:nbuf]
        dbs = rest[nbuf:2 * nbuf]
        rbs = rest[2 * nbuf:3 * nbuf]
        sems = rest[3 * nbuf:]
        six = sems[:_NG]
        sg = sems[_NG:2 * _NG]
        ssc = sems[2 * _NG:]
        c = lax.axis_index("c")
        s = lax.axis_index("s")
        r0 = pl.multiple_of(s * rows, 8)
        pltpu.sync_copy(zeros.at[pl.ds(r0, rows), :], agg_sh.at[pl.ds(r0, rows), :])
        base = c * (e // NC) + s * ep
        plsc.subcore_barrier()

        def off_of(j):
            return pl.multiple_of(base + j * _CA, 8)

        def sl(g, b):
            return g * _NA + b

        def issue_idx(j, g, b):
            off = off_of(j)
            pltpu.async_copy(src_ix.at[pl.ds(off, _CA)], sbs[sl(g, b)], six[g])
            pltpu.async_copy(dst_ix.at[pl.ds(off, _CA)], dbs[sl(g, b)], six[g])

        def wait_idx(j, g, b):
            off = off_of(j)
            pltpu.make_async_copy(
                src_ix.at[pl.ds(off, _CA)], sbs[sl(g, b)], six[g]).wait()
            pltpu.make_async_copy(
                dst_ix.at[pl.ds(off, _CA)], dbs[sl(g, b)], six[g]).wait()

        def body(gg, carry):
            j0 = gg * _NG * _NA
            jg = [j0, j0 + _NA]
            for g in range(_NG):
                for b in range(_NA):
                    issue_idx(jg[g] + b, g, b)
            for g in range(_NG):
                for b in range(_NA):
                    wait_idx(jg[g] + b, g, b)
                for b in range(_NA):
                    pltpu.async_copy(h.at[sbs[sl(g, b)]], rbs[sl(g, b)], sg[g])
            for g in range(_NG):
                for b in range(_NA):
                    pltpu.make_async_copy(
                        h.at[sbs[sl(g, b)]], rbs[sl(g, b)], sg[g]).wait()
                for b in range(_NA):
                    pltpu.async_copy(
                        rbs[sl(g, b)], agg_sh.at[dbs[sl(g, b)]], ssc[g], add=True)
            for g in range(_NG):
                for b in range(_NA):
                    pltpu.make_async_copy(
                        rbs[sl(g, b)], agg_sh.at[dbs[sl(g, b)]], ssc[g]).wait()
            return carry

        lax.fori_loop(0, nchunks // (_NG * _NA), body, 0)
        for j in range(nchunks - nchunks % (_NG * _NA), nchunks):
            off = pl.multiple_of(base + j * _CA, 8)
            pltpu.sync_copy(src_ix.at[pl.ds(off, _CA)], sbs[0])
            pltpu.sync_copy(dst_ix.at[pl.ds(off, _CA)], dbs[0])
            pltpu.async_copy(h.at[sbs[0]], rbs[0], sg[0]).wait()
            pltpu.sync_copy(rbs[0], agg_sh.at[dbs[0]], add=True)
        plsc.subcore_barrier()
        pltpu.sync_copy(agg_sh.at[pl.ds(r0, rows), :], out.at[c, pl.ds(r0, rows), :])

    return agg_kernel


_R = 1000


def _tc1_body(deg2, x, w, dinv_o, h_o):
    deg = deg2[0, :, 0:1] + deg2[1, :, 0:1] + 1.0
    dinv = lax.rsqrt(deg)
    dinv_o[...] = dinv
    h = lax.dot_general(x[...], w[...], (((1,), (1,)), ((), ())),
                        preferred_element_type=jnp.float32)
    h_o[...] = h * dinv


def _tc_mid_body(aggp, hprev, xres, dinv, b, g, be, w, x_o, h_o):
    agg = aggp[0] + aggp[1]
    dv = dinv[...]
    h = dv * (agg + hprev[...]) + b[...][None, :]
    mu = jnp.mean(h, axis=-1, keepdims=True)
    var = jnp.mean((h - mu) ** 2, axis=-1, keepdims=True)
    ln = (h - mu) * lax.rsqrt(var + 1e-5) * g[...][None, :] + be[...][None, :]
    xn = jnp.maximum(ALPHA * ln + (1.0 - ALPHA) * xres[...], 0.0)
    x_o[...] = xn
    h2 = lax.dot_general(xn, w[...], (((1,), (1,)), ((), ())),
                         preferred_element_type=jnp.float32)
    h_o[...] = h2 * dv


def _tc_out_body(aggp, hprev, dinv, b, out_o):
    agg = aggp[0] + aggp[1]
    out_o[...] = dinv[...] * (agg + hprev[...]) + b[...][None, :]


@functools.lru_cache(maxsize=None)
def _make_tc_kernels(n, d):
    grid = (n // _R,)
    bpart = pl.BlockSpec((NC, _R, d), lambda i: (0, i, 0))
    bpcol = pl.BlockSpec((NC, _R, 1), lambda i: (0, i, 0))
    brow = pl.BlockSpec((_R, d), lambda i: (i, 0))
    bcol = pl.BlockSpec((_R, 1), lambda i: (i, 0))
    bvec = pl.BlockSpec((d,), lambda i: (0,))
    bmat = pl.BlockSpec((d, d), lambda i: (0, 0))
    f32 = jnp.float32

    tc1 = pl.pallas_call(
        _tc1_body,
        grid=grid,
        in_specs=[pl.BlockSpec((NC, _R, _DW), lambda i: (0, i, 0)), brow, bmat],
        out_specs=[bcol, brow],
        out_shape=[jax.ShapeDtypeStruct((n, 1), f32),
                   jax.ShapeDtypeStruct((n, d), f32)],
    )
    tc_mid = pl.pallas_call(
        _tc_mid_body,
        grid=grid,
        in_specs=[bpart, brow, brow, bcol, bvec, bvec, bvec, bmat],
        out_specs=[brow, brow],
        out_shape=[jax.ShapeDtypeStruct((n, d), f32),
                   jax.ShapeDtypeStruct((n, d), f32)],
    )
    tc_out = pl.pallas_call(
        _tc_out_body,
        grid=grid,
        in_specs=[bpart, brow, bcol, bvec],
        out_specs=brow,
        out_shape=jax.ShapeDtypeStruct((n, d), f32),
    )
    return tc1, tc_mid, tc_out


def kernel(x, edge_index, W1, b1, g1, be1, W2, b2, g2, be2, W3, b3):
    n, d = x.shape
    e = edge_index.shape[1]
    np_, _ = _pad_rows(n)
    lcm = math.lcm(C, _CA)
    ept = -(-(-(-e // (NC * NS))) // lcm) * lcm
    pe = ept * NC * NS
    deg_k = _make_deg_kernel(n, pe)
    agg_k = _make_agg_kernel(n, d, pe)
    tc1, tc_mid, tc_out = _make_tc_kernels(n, d)

    src_ix = edge_index[0]
    dst_ix = edge_index[1]
    if pe > e:
        pad = pe - e
        src_ix = jnp.concatenate([src_ix, jnp.zeros((pad,), edge_index.dtype)])
        dst_ix = jnp.concatenate(
            [dst_ix, jnp.full((pad,), np_ - 2, edge_index.dtype)])
    zeros = jnp.zeros((np_, d), jnp.float32)
    zeros1 = zeros if _DW == d else jnp.zeros((np_, _DW), jnp.float32)
    ones = jnp.ones((C, _DW), jnp.float32)

    deg2 = deg_k(dst_ix, zeros1, ones)
    dinv, h1 = tc1(deg2, x, W1)
    p1 = agg_k(h1, src_ix, dst_ix, zeros)
    x1, h2 = tc_mid(p1, h1, x, dinv, b1, g1, be1, W2)
    p2 = agg_k(h2, src_ix, dst_ix, zeros)
    x2, h3 = tc_mid(p2, h2, x1, dinv, b2, g2, be2, W3)
    p3 = agg_k(h3, src_ix, dst_ix, zeros)
    return tc_out(p3, h3, dinv, b3)


# speedup vs baseline: 1.2448x; 1.2448x over previous
import functools
import math

import jax
import jax.numpy as jnp
from jax import lax
from jax.experimental import pallas as pl
from jax.experimental.pallas import tpu as pltpu
from jax.experimental.pallas import tpu_sc as plsc

ALPHA = 0.5
NC = 2
NS = 16
C = 80


def _pad_rows(n):
    per = -(-n // NS)
    per = -(-per // 8) * 8
    return per * NS, per


_NB = 4
_DW = 128


@functools.lru_cache(maxsize=None)
def _make_deg_kernel(n, e):
    ep = e // (NC * NS)
    nchunks = ep // C
    np_, rows = _pad_rows(n)
    mesh = plsc.VectorSubcoreMesh(core_axis_name="c", subcore_axis_name="s")

    @functools.partial(
        pl.kernel,
        out_type=jax.ShapeDtypeStruct((NC, np_, _DW), jnp.float32),
        mesh=mesh,
        scratch_types=[
            pltpu.VMEM_SHARED((np_, _DW), jnp.float32),
            pltpu.VMEM((C, _DW), jnp.float32),
        ] + [pltpu.VMEM((C,), jnp.int32) for _ in range(_NB)]
          + [pltpu.SemaphoreType.DMA for _ in range(_NB)],
    )
    def deg_kernel(dst_ix, zeros1, ones, out, deg_sh, onesbuf, *dbsx):
        dbs, sixs = dbsx[:_NB], dbsx[_NB:]
        c = lax.axis_index("c")
        s = lax.axis_index("s")
        r0 = pl.multiple_of(s * rows, 8)
        pltpu.sync_copy(zeros1.at[pl.ds(r0, rows), :], deg_sh.at[pl.ds(r0, rows), :])
        pltpu.sync_copy(ones, onesbuf)
        plsc.subcore_barrier()
        base = c * (e // NC) + s * ep

        six, ssc = sixs[0], sixs[1]

        def group(g, carry):
            j0 = g * _NB
            for b in range(_NB):
                off = pl.multiple_of(base + (j0 + b) * C, 8)
                pltpu.async_copy(dst_ix.at[pl.ds(off, C)], dbs[b], six)
            for b in range(_NB):
                off = pl.multiple_of(base + (j0 + b) * C, 8)
                pltpu.make_async_copy(dst_ix.at[pl.ds(off, C)], dbs[b], six).wait()
            for b in range(_NB):
                pltpu.async_copy(onesbuf, deg_sh.at[dbs[b]], ssc, add=True)
            for b in range(_NB):
                pltpu.make_async_copy(onesbuf, deg_sh.at[dbs[b]], ssc).wait()
            return carry

        lax.fori_loop(0, nchunks // _NB, group, 0)
        for j in range(nchunks - nchunks % _NB, nchunks):
            off = pl.multiple_of(base + j * C, 8)
            pltpu.sync_copy(dst_ix.at[pl.ds(off, C)], dbs[0])
            pltpu.sync_copy(onesbuf, deg_sh.at[dbs[0]], add=True)
        plsc.subcore_barrier()
        pltpu.sync_copy(deg_sh.at[pl.ds(r0, rows), :], out.at[c, pl.ds(r0, rows), :])

    return deg_kernel


_CA = 80
_NG = 1
_NA = 4


@functools.lru_cache(maxsize=None)
def _make_agg_kernel(n, d, e):
    ep = e // (NC * NS)
    nchunks = ep // _CA
    assert ep % _CA == 0
    np_, rows = _pad_rows(n)
    nbuf = _NG * _NA
    mesh = plsc.VectorSubcoreMesh(core_axis_name="c", subcore_axis_name="s")

    @functools.partial(
        pl.kernel,
        out_type=jax.ShapeDtypeStruct((NC, np_, d), jnp.float32),
        mesh=mesh,
        scratch_types=[
            pltpu.VMEM_SHARED((np_, d), jnp.float32),
        ] + [pltpu.VMEM((_CA,), jnp.int32) for _ in range(2 * nbuf)]
          + [pltpu.VMEM((_CA, d), jnp.float32) for _ in range(nbuf)]
          + [pltpu.SemaphoreType.DMA for _ in range(2 * nbuf)],
    )
    def agg_kernel(h, src_ix, dst_ix, zeros, out, agg_sh, *rest):
        sbs = rest[:nbuf]
        dbs = rest[nbuf:2 * nbuf]
        rbs = rest[2 * nbuf:3 * nbuf]
        sems = rest[3 * nbuf:]
        sixs = sems[:nbuf]
        sgs = sems[nbuf:]
        c = lax.axis_index("c")
        s = lax.axis_index("s")
        r0 = pl.multiple_of(s * rows, 8)
        pltpu.sync_copy(zeros.at[pl.ds(r0, rows), :], agg_sh.at[pl.ds(r0, rows), :])
        base = c * (e // NC) + s * ep
        plsc.subcore_barrier()

        assert nchunks % nbuf == 1 and nchunks > nbuf

        def off_of(j):
            jc = jnp.minimum(j, nchunks - 1)
            return pl.multiple_of(base + jc * _CA, 8)

        def issue_idx(j, b):
            off = off_of(j)
            pltpu.async_copy(src_ix.at[pl.ds(off, _CA)], sbs[b], sixs[b])
            pltpu.async_copy(dst_ix.at[pl.ds(off, _CA)], dbs[b], sixs[b])

        def wait_idx(j, b):
            off = off_of(j)
            pltpu.make_async_copy(src_ix.at[pl.ds(off, _CA)], sbs[b], sixs[b]).wait()
            pltpu.make_async_copy(dst_ix.at[pl.ds(off, _CA)], dbs[b], sixs[b]).wait()

        for b in range(nbuf):
            issue_idx(b, b)
        for b in range(2):
            wait_idx(b, b)
            pltpu.async_copy(h.at[sbs[b]], rbs[b], sgs[b])

        def body(jj, carry):
            for b in range(nbuf):
                j = nbuf * jj + b
                b2 = (b + 2) % nbuf
                pltpu.make_async_copy(h.at[sbs[b]], rbs[b], sgs[b]).wait()
                pltpu.sync_copy(rbs[b], agg_sh.at[dbs[b]], add=True)
                issue_idx(j + nbuf, b)
                wait_idx(j + 2, b2)
                pltpu.async_copy(h.at[sbs[b2]], rbs[b2], sgs[b2])
            return carry

        lax.fori_loop(0, (nchunks - 1) // nbuf, body, 0)
        pltpu.make_async_copy(h.at[sbs[0]], rbs[0], sgs[0]).wait()
        pltpu.sync_copy(rbs[0], agg_sh.at[dbs[0]], add=True)
        pltpu.make_async_copy(h.at[sbs[1]], rbs[1], sgs[1]).wait()
        for b in (2, 3):
            wait_idx(nchunks - 1, b)
        plsc.subcore_barrier()
        pltpu.sync_copy(agg_sh.at[pl.ds(r0, rows), :], out.at[c, pl.ds(r0, rows), :])

    return agg_kernel


_R = 1000


def _tc1_body(deg2, x, w, dinv_o, h_o):
    deg = deg2[0, :, 0:1] + deg2[1, :, 0:1] + 1.0
    dinv = lax.rsqrt(deg)
    dinv_o[...] = dinv
    h = lax.dot_general(x[...], w[...], (((1,), (1,)), ((), ())),
                        preferred_element_type=jnp.float32)
    h_o[...] = h * dinv


def _tc_mid_body(aggp, hprev, xres, dinv, b, g, be, w, x_o, h_o):
    agg = aggp[0] + aggp[1]
    dv = dinv[...]
    h = dv * (agg + hprev[...]) + b[...][None, :]
    mu = jnp.mean(h, axis=-1, keepdims=True)
    var = jnp.mean((h - mu) ** 2, axis=-1, keepdims=True)
    ln = (h - mu) * lax.rsqrt(var + 1e-5) * g[...][None, :] + be[...][None, :]
    xn = jnp.maximum(ALPHA * ln + (1.0 - ALPHA) * xres[...], 0.0)
    x_o[...] = xn
    h2 = lax.dot_general(xn, w[...], (((1,), (1,)), ((), ())),
                         preferred_element_type=jnp.float32)
    h_o[...] = h2 * dv


def _tc_out_body(aggp, hprev, dinv, b, out_o):
    agg = aggp[0] + aggp[1]
    out_o[...] = dinv[...] * (agg + hprev[...]) + b[...][None, :]


@functools.lru_cache(maxsize=None)
def _make_tc_kernels(n, d):
    grid = (n // _R,)
    bpart = pl.BlockSpec((NC, _R, d), lambda i: (0, i, 0))
    bpcol = pl.BlockSpec((NC, _R, 1), lambda i: (0, i, 0))
    brow = pl.BlockSpec((_R, d), lambda i: (i, 0))
    bcol = pl.BlockSpec((_R, 1), lambda i: (i, 0))
    bvec = pl.BlockSpec((d,), lambda i: (0,))
    bmat = pl.BlockSpec((d, d), lambda i: (0, 0))
    f32 = jnp.float32

    tc1 = pl.pallas_call(
        _tc1_body,
        grid=grid,
        in_specs=[pl.BlockSpec((NC, _R, _DW), lambda i: (0, i, 0)), brow, bmat],
        out_specs=[bcol, brow],
        out_shape=[jax.ShapeDtypeStruct((n, 1), f32),
                   jax.ShapeDtypeStruct((n, d), f32)],
    )
    tc_mid = pl.pallas_call(
        _tc_mid_body,
        grid=grid,
        in_specs=[bpart, brow, brow, bcol, bvec, bvec, bvec, bmat],
        out_specs=[brow, brow],
        out_shape=[jax.ShapeDtypeStruct((n, d), f32),
                   jax.ShapeDtypeStruct((n, d), f32)],
    )
    tc_out = pl.pallas_call(
        _tc_out_body,
        grid=grid,
        in_specs=[bpart, brow, bcol, bvec],
        out_specs=brow,
        out_shape=jax.ShapeDtypeStruct((n, d), f32),
    )
    return tc1, tc_mid, tc_out


def kernel(x, edge_index, W1, b1, g1, be1, W2, b2, g2, be2, W3, b3):
    n, d = x.shape
    e = edge_index.shape[1]
    np_, _ = _pad_rows(n)
    lcm = math.lcm(C, _CA)
    ept = -(-(-(-e // (NC * NS))) // lcm) * lcm
    pe = ept * NC * NS
    deg_k = _make_deg_kernel(n, pe)
    agg_k = _make_agg_kernel(n, d, pe)
    tc1, tc_mid, tc_out = _make_tc_kernels(n, d)

    src_ix = edge_index[0]
    dst_ix = edge_index[1]
    if pe > e:
        pad = pe - e
        src_ix = jnp.concatenate([src_ix, jnp.zeros((pad,), edge_index.dtype)])
        dst_ix = jnp.concatenate(
            [dst_ix, jnp.full((pad,), np_ - 2, edge_index.dtype)])
    zeros = jnp.zeros((np_, d), jnp.float32)
    zeros1 = zeros if _DW == d else jnp.zeros((np_, _DW), jnp.float32)
    ones = jnp.ones((C, _DW), jnp.float32)

    deg2 = deg_k(dst_ix, zeros1, ones)
    dinv, h1 = tc1(deg2, x, W1)
    p1 = agg_k(h1, src_ix, dst_ix, zeros)
    x1, h2 = tc_mid(p1, h1, x, dinv, b1, g1, be1, W2)
    p2 = agg_k(h2, src_ix, dst_ix, zeros)
    x2, h3 = tc_mid(p2, h2, x1, dinv, b2, g2, be2, W3)
    p3 = agg_k(h3, src_ix, dst_ix, zeros)
    return tc_out(p3, h3, dinv, b3)


# speedup vs baseline: 1.2454x; 1.0005x over previous
import functools
import math

import jax
import jax.numpy as jnp
from jax import lax
from jax.experimental import pallas as pl
from jax.experimental.pallas import tpu as pltpu
from jax.experimental.pallas import tpu_sc as plsc

ALPHA = 0.5
NC = 2
NS = 16
C = 80


def _pad_rows(n):
    per = -(-n // NS)
    per = -(-per // 8) * 8
    return per * NS, per


_NB = 4
_DW = 128


@functools.lru_cache(maxsize=None)
def _make_deg_kernel(n, e):
    ep = e // (NC * NS)
    nchunks = ep // C
    np_, rows = _pad_rows(n)
    mesh = plsc.VectorSubcoreMesh(core_axis_name="c", subcore_axis_name="s")

    @functools.partial(
        pl.kernel,
        out_type=jax.ShapeDtypeStruct((NC, np_, _DW), jnp.float32),
        mesh=mesh,
        scratch_types=[
            pltpu.VMEM_SHARED((np_, _DW), jnp.float32),
            pltpu.VMEM((C, _DW), jnp.float32),
        ] + [pltpu.VMEM((C,), jnp.int32) for _ in range(_NB)]
          + [pltpu.SemaphoreType.DMA for _ in range(_NB)],
    )
    def deg_kernel(dst_ix, zeros1, ones, out, deg_sh, onesbuf, *dbsx):
        dbs, sixs = dbsx[:_NB], dbsx[_NB:]
        c = lax.axis_index("c")
        s = lax.axis_index("s")
        r0 = pl.multiple_of(s * rows, 8)
        pltpu.sync_copy(zeros1.at[pl.ds(r0, rows), :], deg_sh.at[pl.ds(r0, rows), :])
        pltpu.sync_copy(ones, onesbuf)
        plsc.subcore_barrier()
        base = c * (e // NC) + s * ep

        six, ssc = sixs[0], sixs[1]

        def group(g, carry):
            j0 = g * _NB
            for b in range(_NB):
                off = pl.multiple_of(base + (j0 + b) * C, 8)
                pltpu.async_copy(dst_ix.at[pl.ds(off, C)], dbs[b], six)
            for b in range(_NB):
                off = pl.multiple_of(base + (j0 + b) * C, 8)
                pltpu.make_async_copy(dst_ix.at[pl.ds(off, C)], dbs[b], six).wait()
            for b in range(_NB):
                pltpu.async_copy(onesbuf, deg_sh.at[dbs[b]], ssc, add=True)
            for b in range(_NB):
                pltpu.make_async_copy(onesbuf, deg_sh.at[dbs[b]], ssc).wait()
            return carry

        lax.fori_loop(0, nchunks // _NB, group, 0)
        for j in range(nchunks - nchunks % _NB, nchunks):
            off = pl.multiple_of(base + j * C, 8)
            pltpu.sync_copy(dst_ix.at[pl.ds(off, C)], dbs[0])
            pltpu.sync_copy(onesbuf, deg_sh.at[dbs[0]], add=True)
        plsc.subcore_barrier()
        pltpu.sync_copy(deg_sh.at[pl.ds(r0, rows), :], out.at[c, pl.ds(r0, rows), :])

    return deg_kernel


_CA = 80
_NP = 4


@functools.lru_cache(maxsize=None)
def _make_agg_kernel(n, d, e):
    ep = e // (NC * NS)
    nchunks = ep // _CA
    assert ep % _CA == 0
    np_, rows = _pad_rows(n)
    nbuf = _NP
    mesh = plsc.VectorSubcoreMesh(core_axis_name="c", subcore_axis_name="s")

    @functools.partial(
        pl.kernel,
        out_type=jax.ShapeDtypeStruct((NC, np_, d), jnp.float32),
        mesh=mesh,
        scratch_types=[
            pltpu.VMEM_SHARED((np_, d), jnp.float32),
        ] + [pltpu.VMEM((_CA,), jnp.int32) for _ in range(2 * nbuf)]
          + [pltpu.VMEM((_CA, d), jnp.float32) for _ in range(nbuf)]
          + [pltpu.SemaphoreType.DMA for _ in range(2 * nbuf)],
    )
    def agg_kernel(h, src_ix, dst_ix, zeros, out, agg_sh, *rest):
        sbs = rest[:nbuf]
        dbs = rest[nbuf:2 * nbuf]
        rbs = rest[2 * nbuf:3 * nbuf]
        sems = rest[3 * nbuf:]
        sixs = sems[:nbuf]
        sgs = sems[nbuf:]
        c = lax.axis_index("c")
        s = lax.axis_index("s")
        r0 = pl.multiple_of(s * rows, 8)
        pltpu.sync_copy(zeros.at[pl.ds(r0, rows), :], agg_sh.at[pl.ds(r0, rows), :])
        base = c * (e // NC) + s * ep
        plsc.subcore_barrier()

        assert nchunks % nbuf == 1 and nchunks > nbuf

        def off_of(j):
            jc = jnp.minimum(j, nchunks - 1)
            return pl.multiple_of(base + jc * _CA, 8)

        def issue_idx(j, b):
            off = off_of(j)
            pltpu.async_copy(src_ix.at[pl.ds(off, _CA)], sbs[b], sixs[b])
            pltpu.async_copy(dst_ix.at[pl.ds(off, _CA)], dbs[b], sixs[b])

        def wait_idx(j, b):
            off = off_of(j)
            pltpu.make_async_copy(src_ix.at[pl.ds(off, _CA)], sbs[b], sixs[b]).wait()
            pltpu.make_async_copy(dst_ix.at[pl.ds(off, _CA)], dbs[b], sixs[b]).wait()

        for b in range(nbuf):
            issue_idx(b, b)
        for b in range(2):
            wait_idx(b, b)
            pltpu.async_copy(h.at[sbs[b]], rbs[b], sgs[b])

        def body(jj, carry):
            for b in range(nbuf):
                j = nbuf * jj + b
                b2 = (b + 2) % nbuf
                pltpu.make_async_copy(h.at[sbs[b]], rbs[b], sgs[b]).wait()
                pltpu.sync_copy(rbs[b], agg_sh.at[dbs[b]], add=True)
                issue_idx(j + nbuf, b)
                wait_idx(j + 2, b2)
                pltpu.async_copy(h.at[sbs[b2]], rbs[b2], sgs[b2])
            return carry

        lax.fori_loop(0, (nchunks - 1) // nbuf, body, 0)
        pltpu.make_async_copy(h.at[sbs[0]], rbs[0], sgs[0]).wait()
        pltpu.sync_copy(rbs[0], agg_sh.at[dbs[0]], add=True)
        pltpu.make_async_copy(h.at[sbs[1]], rbs[1], sgs[1]).wait()
        for b in (2, 3):
            wait_idx(nchunks - 1, b)
        plsc.subcore_barrier()
        pltpu.sync_copy(agg_sh.at[pl.ds(r0, rows), :], out.at[c, pl.ds(r0, rows), :])

    return agg_kernel


_R = 1000


def _tc1_body(deg2, x, w, dinv_o, h_o):
    deg = deg2[0, :, 0:1] + deg2[1, :, 0:1] + 1.0
    dinv = lax.rsqrt(deg)
    dinv_o[...] = dinv
    h = lax.dot_general(x[...], w[...], (((1,), (1,)), ((), ())),
                        preferred_element_type=jnp.float32)
    h_o[...] = h * dinv


def _tc_mid_body(aggp, hprev, xres, dinv, b, g, be, w, x_o, h_o):
    agg = aggp[0] + aggp[1]
    dv = dinv[...]
    h = dv * (agg + hprev[...]) + b[...][None, :]
    mu = jnp.mean(h, axis=-1, keepdims=True)
    var = jnp.mean((h - mu) ** 2, axis=-1, keepdims=True)
    ln = (h - mu) * lax.rsqrt(var + 1e-5) * g[...][None, :] + be[...][None, :]
    xn = jnp.maximum(ALPHA * ln + (1.0 - ALPHA) * xres[...], 0.0)
    x_o[...] = xn
    h2 = lax.dot_general(xn, w[...], (((1,), (1,)), ((), ())),
                         preferred_element_type=jnp.float32)
    h_o[...] = h2 * dv


def _tc_out_body(aggp, hprev, dinv, b, out_o):
    agg = aggp[0] + aggp[1]
    out_o[...] = dinv[...] * (agg + hprev[...]) + b[...][None, :]


@functools.lru_cache(maxsize=None)
def _make_tc_kernels(n, d):
    grid = (n // _R,)
    bpart = pl.BlockSpec((NC, _R, d), lambda i: (0, i, 0))
    brow = pl.BlockSpec((_R, d), lambda i: (i, 0))
    bcol = pl.BlockSpec((_R, 1), lambda i: (i, 0))
    bvec = pl.BlockSpec((d,), lambda i: (0,))
    bmat = pl.BlockSpec((d, d), lambda i: (0, 0))
    f32 = jnp.float32

    tc1 = pl.pallas_call(
        _tc1_body,
        grid=grid,
        in_specs=[pl.BlockSpec((NC, _R, _DW), lambda i: (0, i, 0)), brow, bmat],
        out_specs=[bcol, brow],
        out_shape=[jax.ShapeDtypeStruct((n, 1), f32),
                   jax.ShapeDtypeStruct((n, d), f32)],
    )
    tc_mid = pl.pallas_call(
        _tc_mid_body,
        grid=grid,
        in_specs=[bpart, brow, brow, bcol, bvec, bvec, bvec, bmat],
        out_specs=[brow, brow],
        out_shape=[jax.ShapeDtypeStruct((n, d), f32),
                   jax.ShapeDtypeStruct((n, d), f32)],
    )
    tc_out = pl.pallas_call(
        _tc_out_body,
        grid=grid,
        in_specs=[bpart, brow, bcol, bvec],
        out_specs=brow,
        out_shape=jax.ShapeDtypeStruct((n, d), f32),
    )
    return tc1, tc_mid, tc_out


def kernel(x, edge_index, W1, b1, g1, be1, W2, b2, g2, be2, W3, b3):
    n, d = x.shape
    e = edge_index.shape[1]
    np_, _ = _pad_rows(n)
    lcm = math.lcm(C, _CA)
    ept = -(-(-(-e // (NC * NS))) // lcm) * lcm
    pe = ept * NC * NS
    deg_k = _make_deg_kernel(n, pe)
    agg_k = _make_agg_kernel(n, d, pe)
    tc1, tc_mid, tc_out = _make_tc_kernels(n, d)

    src_ix = edge_index[0]
    dst_ix = edge_index[1]
    if pe > e:
        pad = pe - e
        src_ix = jnp.concatenate([src_ix, jnp.zeros((pad,), edge_index.dtype)])
        dst_ix = jnp.concatenate(
            [dst_ix, jnp.full((pad,), np_ - 2, edge_index.dtype)])
    zeros = jnp.zeros((np_, d), jnp.float32)
    zeros1 = zeros if _DW == d else jnp.zeros((np_, _DW), jnp.float32)
    ones = jnp.ones((C, _DW), jnp.float32)

    deg2 = deg_k(dst_ix, zeros1, ones)
    dinv, h1 = tc1(deg2, x, W1)
    p1 = agg_k(h1, src_ix, dst_ix, zeros)
    x1, h2 = tc_mid(p1, h1, x, dinv, b1, g1, be1, W2)
    p2 = agg_k(h2, src_ix, dst_ix, zeros)
    x2, h3 = tc_mid(p2, h2, x1, dinv, b2, g2, be2, W3)
    p3 = agg_k(h3, src_ix, dst_ix, zeros)
    return tc_out(p3, h3, dinv, b3)
